# baseline (speedup 1.0000x reference)
import functools

import jax
import jax.numpy as jnp
from jax import lax
from jax.experimental import pallas as pl
from jax.experimental.pallas import tpu as pltpu
from jax.experimental.pallas import tpu_sc as plsc

N_ENT = 10000
H = 256
NSL = H // 16
NW = 32
NCHUNK = 2
NN = 160
NPAD = NW * NCHUNK * NN
EB = 1600
GB = 32
SRC_BITS = 14
SRC_MASK = (1 << SRC_BITS) - 1


def _make_sc_neigh(n_edges):
  assert n_edges % (2 * EB) == 0
  n_blocks = n_edges // EB
  npairs = n_blocks // 2
  mesh = plsc.VectorSubcoreMesh(core_axis_name="c", subcore_axis_name="s")

  @functools.partial(
      pl.kernel,
      mesh=mesh,
      out_type=jax.ShapeDtypeStruct((NPAD, H), jnp.float32),
      scratch_types=[
          pltpu.VMEM((NN, H), jnp.float32),
          pltpu.VMEM((NN, H), jnp.float32),
          pltpu.SMEM((NN,), jnp.float32),
          pltpu.SMEM((NN,), jnp.float32),
          pltpu.SMEM((2,), jnp.int32),
          pltpu.VMEM((EB,), jnp.int32),
          pltpu.VMEM((EB,), jnp.int32),
          pltpu.VMEM((EB,), jnp.int32),
          pltpu.VMEM((EB,), jnp.int32),
          pltpu.VMEM((EB + 32,), jnp.int32),
          pltpu.VMEM((EB + 32,), jnp.int32),
          pltpu.VMEM((EB + 32,), jnp.int32),
          pltpu.VMEM((EB + 32,), jnp.int32),
          pltpu.VMEM((GB,), jnp.int32),
          pltpu.VMEM((GB,), jnp.int32),
          pltpu.VMEM((GB,), jnp.int32),
          pltpu.VMEM((GB,), jnp.int32),
          pltpu.VMEM((GB, H), jnp.float32),
          pltpu.VMEM((GB, H), jnp.float32),
          pltpu.VMEM((GB, H), jnp.float32),
          pltpu.VMEM((GB, H), jnp.float32),
          pltpu.SemaphoreType.DMA,
          pltpu.SemaphoreType.DMA,
          pltpu.SemaphoreType.DMA,
          pltpu.SemaphoreType.DMA,
      ],
      compiler_params=pltpu.CompilerParams(needs_layout_passes=False),
  )
  def k(ent_hbm, rel_hbm, pk_hbm, dst_hbm, out_hbm,
        dstv, acc, m_l, s_l, cnt_s,
        pblk0, dblk0, pblk1, dblk1,
        lpk0, ldst0, lpk1, ldst1,
        sidx0, ridx0, sidx1, ridx1,
        srows0, rrows0, srows1, rrows1,
        bsem0, bsem1, gsem0, gsem1):
    wid = lax.axis_index("s") * 2 + lax.axis_index("c")
    zi16 = jnp.zeros((16,), jnp.int32)
    zf16 = jnp.zeros((16,), jnp.float32)

    def issue_blk(b, pb, db, sem):
      bo = b * EB
      pltpu.async_copy(pk_hbm.at[pl.ds(bo, EB)], pb, sem)
      pltpu.async_copy(dst_hbm.at[pl.ds(bo, EB)], db, sem)

    def wait_blk(b, pb, db, sem):
      bo = b * EB
      pltpu.make_async_copy(pk_hbm.at[pl.ds(bo, EB)], pb, sem).wait()
      pltpu.make_async_copy(dst_hbm.at[pl.ds(bo, EB)], db, sem).wait()

    def unpack(lpk, base, sidx, ridx):
      for u in range(GB // 16):
        v = lpk[pl.ds(base + 16 * u, 16)]
        sidx[pl.ds(16 * u, 16)] = v & SRC_MASK
        ridx[pl.ds(16 * u, 16)] = lax.shift_right_logical(v, SRC_BITS)

    def issue_gather(sidx, ridx, srows, rrows, gsem):
      pltpu.async_copy(ent_hbm.at[sidx], srows, gsem)
      pltpu.async_copy(rel_hbm.at[ridx], rrows, gsem)

    def wait_gather(sidx, ridx, srows, rrows, gsem):
      pltpu.make_async_copy(ent_hbm.at[sidx], srows, gsem).wait()
      pltpu.make_async_copy(rel_hbm.at[ridx], rrows, gsem).wait()

    def chunk_body(c, _):
      lo = (wid * NCHUNK + c) * NN
      pltpu.sync_copy(ent_hbm.at[pl.ds(lo, NN), :], dstv)

      def zrow(i, _):
        for t in range(NSL):
          acc[i, pl.ds(16 * t, 16)] = zf16
        m_l[i] = jnp.float32(-3.4e38)
        s_l[i] = jnp.float32(0.0)
        return 0
      lax.fori_loop(0, NN, zrow, 0)

      def scan(pb, db, lpk, ldstl, slot):
        def body(jv, cnt):
          sl = pl.ds(jv * 16, 16)
          vdl = db[sl] - lo
          msk = vdl.astype(jnp.uint32) < jnp.uint32(NN)
          incl = plsc.cumsum(jnp.where(msk, 1, 0).astype(jnp.int32))
          pos = cnt + incl - 1
          plsc.store_scatter(lpk, [pos], pb[sl], mask=msk)
          plsc.store_scatter(ldstl, [pos], vdl, mask=msk)
          return cnt + incl[15]
        cnt = lax.fori_loop(0, EB // 16, body, jnp.int32(0))
        lpk[pl.ds(cnt, 16)] = zi16
        lpk[pl.ds(cnt + 16, 16)] = zi16
        cnt_s[slot] = cnt

      def edge_loop(ne, base, ldstl, srows, rrows):
        def edge_body(j, _):
          ldj = ldstl[pl.ds(base + j, 16)][0]
          dacc = jnp.zeros((16,), jnp.float32)
          for t in range(NSL):
            hs = pl.ds(16 * t, 16)
            dacc = dacc + (srows[j, hs] + rrows[j, hs]) * dstv[ldj, hs]
          norm = jnp.sum(dacc)
          m_old = m_l[ldj]
          s_old = s_l[ldj]

          @pl.when(norm > m_old)
          def _():
            corr = jnp.max(jnp.exp(jnp.full((16,), m_old - norm, jnp.float32)))
            s_l[ldj] = s_old * corr + jnp.float32(1.0)
            m_l[ldj] = norm
            for t in range(NSL):
              hs = pl.ds(16 * t, 16)
              acc[ldj, hs] = (acc[ldj, hs] * corr
                              + (srows[j, hs] + rrows[j, hs]))

          @pl.when(jnp.logical_not(norm > m_old))
          def _():
            w = jnp.max(jnp.exp(jnp.full((16,), norm - m_old, jnp.float32)))
            s_l[ldj] = s_old + w
            for t in range(NSL):
              hs = pl.ds(16 * t, 16)
              acc[ldj, hs] = (acc[ldj, hs]
                              + (srows[j, hs] + rrows[j, hs]) * w)
          return 0
        lax.fori_loop(0, ne, edge_body, 0)

      def process(lpk, ldstl, sidx, ridx, srows, rrows, gsem, slot):
        cnt = cnt_s[slot]
        wait_gather(sidx, ridx, srows, rrows, gsem)
        edge_loop(jnp.minimum(cnt, GB), 0, ldstl, srows, rrows)
        nbat = (cnt + (GB - 1)) // GB

        def extra(bb, _):
          base = bb * GB
          unpack(lpk, base, sidx, ridx)
          issue_gather(sidx, ridx, srows, rrows, gsem)
          wait_gather(sidx, ridx, srows, rrows, gsem)
          edge_loop(jnp.minimum(cnt - base, GB), base, ldstl, srows, rrows)
          return 0
        lax.fori_loop(1, nbat, extra, 0)

      def scan_and_issue(pb, db, lpk, ldstl, sidx, ridx, srows, rrows,
                         gsem, slot):
        scan(pb, db, lpk, ldstl, slot)
        unpack(lpk, 0, sidx, ridx)
        issue_gather(sidx, ridx, srows, rrows, gsem)

      issue_blk(0, pblk0, dblk0, bsem0)
      issue_blk(1, pblk1, dblk1, bsem1)
      wait_blk(0, pblk0, dblk0, bsem0)
      scan_and_issue(pblk0, dblk0, lpk0, ldst0, sidx0, ridx0,
                     srows0, rrows0, gsem0, 0)
      issue_blk(2, pblk0, dblk0, bsem0)

      def pair_body(i, _):
        b0 = 2 * i
        wait_blk(b0 + 1, pblk1, dblk1, bsem1)
        scan_and_issue(pblk1, dblk1, lpk1, ldst1, sidx1, ridx1,
                       srows1, rrows1, gsem1, 1)

        @pl.when(b0 + 3 < n_blocks)
        def _():
          issue_blk(b0 + 3, pblk1, dblk1, bsem1)

        process(lpk0, ldst0, sidx0, ridx0, srows0, rrows0, gsem0, 0)

        @pl.when(b0 + 2 < n_blocks)
        def _():
          wait_blk(b0 + 2, pblk0, dblk0, bsem0)
          scan_and_issue(pblk0, dblk0, lpk0, ldst0, sidx0, ridx0,
                         srows0, rrows0, gsem0, 0)

          @pl.when(b0 + 4 < n_blocks)
          def _():
            issue_blk(b0 + 4, pblk0, dblk0, bsem0)

        process(lpk1, ldst1, sidx1, ridx1, srows1, rrows1, gsem1, 1)
        return 0
      lax.fori_loop(0, npairs, pair_body, 0)

      def fin(i, _):
        sv = jnp.full((16,), s_l[i], jnp.float32) + jnp.float32(1e-16)
        invv = jnp.float32(1.0) / sv
        for t in range(NSL):
          hs = pl.ds(16 * t, 16)
          acc[i, hs] = acc[i, hs] * invv
        return 0
      lax.fori_loop(0, NN, fin, 0)
      pltpu.sync_copy(acc, out_hbm.at[pl.ds(lo, NN), :])
      return 0
    lax.fori_loop(0, NCHUNK, chunk_body, 0)

  return k


def _mm_tanh_body(neigh_ref, w_ref, out_ref):
  out_ref[...] = jnp.tanh(
      jnp.dot(neigh_ref[...], w_ref[...], preferred_element_type=jnp.float32))


def kernel(ent_emb, rel_emb, edge_index, rel_id, neigh_w):
  src = edge_index[0].astype(jnp.int32)
  dst = edge_index[1].astype(jnp.int32)
  rel = rel_id.astype(jnp.int32)
  packed = src | (rel << SRC_BITS)
  n_edges = src.shape[0]

  ent_pad = jnp.concatenate(
      [ent_emb, jnp.zeros((NPAD - N_ENT, H), jnp.float32)], axis=0)

  neigh = _make_sc_neigh(n_edges)(ent_pad, rel_emb, packed, dst)
  neigh = neigh[:N_ENT]

  blk = 1000
  out = pl.pallas_call(
      _mm_tanh_body,
      grid=(N_ENT // blk,),
      in_specs=[
          pl.BlockSpec((blk, H), lambda i: (i, 0)),
          pl.BlockSpec((H, H), lambda i: (0, 0)),
      ],
      out_specs=pl.BlockSpec((blk, H), lambda i: (i, 0)),
      out_shape=jax.ShapeDtypeStruct((N_ENT, H), jnp.float32),
  )(neigh, neigh_w)
  return out


# speedup vs baseline: 1.0065x; 1.0065x over previous
import functools

import jax
import jax.numpy as jnp
from jax import lax
from jax.experimental import pallas as pl
from jax.experimental.pallas import tpu as pltpu
from jax.experimental.pallas import tpu_sc as plsc

N_ENT = 10000
H = 256
NSL = H // 16
NW = 32
NCHUNK = 2
NN = 160
NPAD = NW * NCHUNK * NN
EB = 1600
GB = 32
SRC_BITS = 14
SRC_MASK = (1 << SRC_BITS) - 1


def _make_sc_neigh(n_edges):
  assert n_edges % (2 * EB) == 0
  n_blocks = n_edges // EB
  npairs = n_blocks // 2
  mesh = plsc.VectorSubcoreMesh(core_axis_name="c", subcore_axis_name="s")

  @functools.partial(
      pl.kernel,
      mesh=mesh,
      out_type=jax.ShapeDtypeStruct((NPAD, H), jnp.float32),
      scratch_types=[
          pltpu.VMEM((NN, H), jnp.float32),
          pltpu.VMEM((NN, H), jnp.float32),
          pltpu.SMEM((NN,), jnp.float32),
          pltpu.SMEM((NN,), jnp.float32),
          pltpu.SMEM((2,), jnp.int32),
          pltpu.VMEM((EB,), jnp.int32),
          pltpu.VMEM((EB,), jnp.int32),
          pltpu.VMEM((EB,), jnp.int32),
          pltpu.VMEM((EB,), jnp.int32),
          pltpu.VMEM((EB + 32,), jnp.int32),
          pltpu.VMEM((EB + 32,), jnp.int32),
          pltpu.VMEM((EB + 32,), jnp.int32),
          pltpu.VMEM((EB + 32,), jnp.int32),
          pltpu.VMEM((GB,), jnp.int32),
          pltpu.VMEM((GB,), jnp.int32),
          pltpu.VMEM((GB,), jnp.int32),
          pltpu.VMEM((GB,), jnp.int32),
          pltpu.VMEM((GB, H), jnp.float32),
          pltpu.VMEM((GB, H), jnp.float32),
          pltpu.VMEM((GB, H), jnp.float32),
          pltpu.VMEM((GB, H), jnp.float32),
          pltpu.SemaphoreType.DMA,
          pltpu.SemaphoreType.DMA,
          pltpu.SemaphoreType.DMA,
          pltpu.SemaphoreType.DMA,
      ],
      compiler_params=pltpu.CompilerParams(needs_layout_passes=False),
  )
  def k(ent_hbm, rel_hbm, pk_hbm, dst_hbm, out_hbm,
        dstv, acc, m_l, s_l, cnt_s,
        pblk0, dblk0, pblk1, dblk1,
        lpk0, ldst0, lpk1, ldst1,
        sidx0, ridx0, sidx1, ridx1,
        srows0, rrows0, srows1, rrows1,
        bsem0, bsem1, gsem0, gsem1):
    wid = lax.axis_index("s") * 2 + lax.axis_index("c")
    zi16 = jnp.zeros((16,), jnp.int32)
    zf16 = jnp.zeros((16,), jnp.float32)

    def issue_blk(b, pb, db, sem):
      bo = b * EB
      pltpu.async_copy(pk_hbm.at[pl.ds(bo, EB)], pb, sem)
      pltpu.async_copy(dst_hbm.at[pl.ds(bo, EB)], db, sem)

    def wait_blk(b, pb, db, sem):
      bo = b * EB
      pltpu.make_async_copy(pk_hbm.at[pl.ds(bo, EB)], pb, sem).wait()
      pltpu.make_async_copy(dst_hbm.at[pl.ds(bo, EB)], db, sem).wait()

    def unpack(lpk, base, sidx, ridx):
      for u in range(GB // 16):
        v = lpk[pl.ds(base + 16 * u, 16)]
        sidx[pl.ds(16 * u, 16)] = v & SRC_MASK
        ridx[pl.ds(16 * u, 16)] = lax.shift_right_logical(v, SRC_BITS)

    def issue_gather(sidx, ridx, srows, rrows, gsem):
      pltpu.async_copy(ent_hbm.at[sidx], srows, gsem)
      pltpu.async_copy(rel_hbm.at[ridx], rrows, gsem)

    def wait_gather(sidx, ridx, srows, rrows, gsem):
      pltpu.make_async_copy(ent_hbm.at[sidx], srows, gsem).wait()
      pltpu.make_async_copy(rel_hbm.at[ridx], rrows, gsem).wait()

    def chunk_body(c, _):
      lo = (wid * NCHUNK + c) * NN
      pltpu.sync_copy(ent_hbm.at[pl.ds(lo, NN), :], dstv)

      def zrow(i, _):
        for t in range(NSL):
          acc[i, pl.ds(16 * t, 16)] = zf16
        m_l[i] = jnp.float32(-3.4e38)
        s_l[i] = jnp.float32(0.0)
        return 0
      lax.fori_loop(0, NN, zrow, 0)

      def scan(pb, db, lpk, ldstl, slot):
        def body(jv, cnt):
          sl = pl.ds(jv * 16, 16)
          vdl = db[sl] - lo
          msk = vdl.astype(jnp.uint32) < jnp.uint32(NN)
          incl = plsc.cumsum(jnp.where(msk, 1, 0).astype(jnp.int32))
          pos = cnt + incl - 1
          plsc.store_scatter(lpk, [pos], pb[sl], mask=msk)
          plsc.store_scatter(ldstl, [pos], vdl, mask=msk)
          return cnt + incl[15]
        cnt = lax.fori_loop(0, EB // 16, body, jnp.int32(0))
        lpk[pl.ds(cnt, 16)] = zi16
        lpk[pl.ds(cnt + 16, 16)] = zi16
        cnt_s[slot] = cnt

      def edge_loop(ne, base, ldstl, srows, rrows):
        def edge_body(j, _):
          ldj = ldstl[pl.ds(base + j, 16)][0]
          dp = [jnp.zeros((16,), jnp.float32) for _ in range(4)]
          for t in range(NSL):
            hs = pl.ds(16 * t, 16)
            dp[t % 4] = dp[t % 4] + (srows[j, hs] + rrows[j, hs]) * dstv[ldj, hs]
          norm = jnp.sum((dp[0] + dp[1]) + (dp[2] + dp[3]))
          m_old = m_l[ldj]
          s_old = s_l[ldj]
          m_new = jnp.maximum(m_old, norm)
          w16 = jnp.exp(jnp.full((16,), norm - m_new, jnp.float32))
          corr16 = jnp.exp(jnp.full((16,), m_old - m_new, jnp.float32))
          s_l[ldj] = s_old * corr16[0] + w16[0]
          m_l[ldj] = m_new
          for t in range(NSL):
            hs = pl.ds(16 * t, 16)
            acc[ldj, hs] = (acc[ldj, hs] * corr16
                            + (srows[j, hs] + rrows[j, hs]) * w16)
          return 0
        lax.fori_loop(0, ne, edge_body, 0)

      def process(lpk, ldstl, sidx, ridx, srows, rrows, gsem, slot):
        cnt = cnt_s[slot]
        wait_gather(sidx, ridx, srows, rrows, gsem)
        edge_loop(jnp.minimum(cnt, GB), 0, ldstl, srows, rrows)
        nbat = (cnt + (GB - 1)) // GB

        def extra(bb, _):
          base = bb * GB
          unpack(lpk, base, sidx, ridx)
          issue_gather(sidx, ridx, srows, rrows, gsem)
          wait_gather(sidx, ridx, srows, rrows, gsem)
          edge_loop(jnp.minimum(cnt - base, GB), base, ldstl, srows, rrows)
          return 0
        lax.fori_loop(1, nbat, extra, 0)

      def scan_and_issue(pb, db, lpk, ldstl, sidx, ridx, srows, rrows,
                         gsem, slot):
        scan(pb, db, lpk, ldstl, slot)
        unpack(lpk, 0, sidx, ridx)
        issue_gather(sidx, ridx, srows, rrows, gsem)

      issue_blk(0, pblk0, dblk0, bsem0)
      issue_blk(1, pblk1, dblk1, bsem1)
      wait_blk(0, pblk0, dblk0, bsem0)
      scan_and_issue(pblk0, dblk0, lpk0, ldst0, sidx0, ridx0,
                     srows0, rrows0, gsem0, 0)
      issue_blk(2, pblk0, dblk0, bsem0)

      def pair_body(i, _):
        b0 = 2 * i
        wait_blk(b0 + 1, pblk1, dblk1, bsem1)
        scan_and_issue(pblk1, dblk1, lpk1, ldst1, sidx1, ridx1,
                       srows1, rrows1, gsem1, 1)

        @pl.when(b0 + 3 < n_blocks)
        def _():
          issue_blk(b0 + 3, pblk1, dblk1, bsem1)

        process(lpk0, ldst0, sidx0, ridx0, srows0, rrows0, gsem0, 0)

        @pl.when(b0 + 2 < n_blocks)
        def _():
          wait_blk(b0 + 2, pblk0, dblk0, bsem0)
          scan_and_issue(pblk0, dblk0, lpk0, ldst0, sidx0, ridx0,
                         srows0, rrows0, gsem0, 0)

          @pl.when(b0 + 4 < n_blocks)
          def _():
            issue_blk(b0 + 4, pblk0, dblk0, bsem0)

        process(lpk1, ldst1, sidx1, ridx1, srows1, rrows1, gsem1, 1)
        return 0
      lax.fori_loop(0, npairs, pair_body, 0)

      def fin(i, _):
        sv = jnp.full((16,), s_l[i], jnp.float32) + jnp.float32(1e-16)
        invv = jnp.float32(1.0) / sv
        for t in range(NSL):
          hs = pl.ds(16 * t, 16)
          acc[i, hs] = acc[i, hs] * invv
        return 0
      lax.fori_loop(0, NN, fin, 0)
      pltpu.sync_copy(acc, out_hbm.at[pl.ds(lo, NN), :])
      return 0
    lax.fori_loop(0, NCHUNK, chunk_body, 0)

  return k


def _mm_tanh_body(neigh_ref, w_ref, out_ref):
  out_ref[...] = jnp.tanh(
      jnp.dot(neigh_ref[...], w_ref[...], preferred_element_type=jnp.float32))


def kernel(ent_emb, rel_emb, edge_index, rel_id, neigh_w):
  src = edge_index[0].astype(jnp.int32)
  dst = edge_index[1].astype(jnp.int32)
  rel = rel_id.astype(jnp.int32)
  packed = src | (rel << SRC_BITS)
  n_edges = src.shape[0]

  ent_pad = jnp.concatenate(
      [ent_emb, jnp.zeros((NPAD - N_ENT, H), jnp.float32)], axis=0)

  neigh = _make_sc_neigh(n_edges)(ent_pad, rel_emb, packed, dst)
  neigh = neigh[:N_ENT]

  blk = 1000
  out = pl.pallas_call(
      _mm_tanh_body,
      grid=(N_ENT // blk,),
      in_specs=[
          pl.BlockSpec((blk, H), lambda i: (i, 0)),
          pl.BlockSpec((H, H), lambda i: (0, 0)),
      ],
      out_specs=pl.BlockSpec((blk, H), lambda i: (i, 0)),
      out_shape=jax.ShapeDtypeStruct((N_ENT, H), jnp.float32),
  )(neigh, neigh_w)
  return out


# speedup vs baseline: 1.0152x; 1.0087x over previous
import functools

import jax
import jax.numpy as jnp
from jax import lax
from jax.experimental import pallas as pl
from jax.experimental.pallas import tpu as pltpu
from jax.experimental.pallas import tpu_sc as plsc

N_ENT = 10000
H = 256
NSL = H // 16
NW = 32
NCHUNK = 2
NN = 160
NPAD = NW * NCHUNK * NN
EB = 1600
GB = 32
SRC_BITS = 14
SRC_MASK = (1 << SRC_BITS) - 1


def _make_sc_neigh(n_edges):
  assert n_edges % (2 * EB) == 0
  n_blocks = n_edges // EB
  npairs = n_blocks // 2
  mesh = plsc.VectorSubcoreMesh(core_axis_name="c", subcore_axis_name="s")

  @functools.partial(
      pl.kernel,
      mesh=mesh,
      out_type=jax.ShapeDtypeStruct((NPAD, H), jnp.float32),
      scratch_types=[
          pltpu.VMEM((NN, H), jnp.float32),
          pltpu.VMEM((NN, H), jnp.float32),
          pltpu.SMEM((NN,), jnp.float32),
          pltpu.SMEM((NN,), jnp.float32),
          pltpu.SMEM((2,), jnp.int32),
          pltpu.VMEM((EB,), jnp.int32),
          pltpu.VMEM((EB,), jnp.int32),
          pltpu.VMEM((EB,), jnp.int32),
          pltpu.VMEM((EB,), jnp.int32),
          pltpu.VMEM((EB + 32,), jnp.int32),
          pltpu.VMEM((EB + 32,), jnp.int32),
          pltpu.VMEM((EB + 32,), jnp.int32),
          pltpu.VMEM((EB + 32,), jnp.int32),
          pltpu.VMEM((GB,), jnp.int32),
          pltpu.VMEM((GB,), jnp.int32),
          pltpu.VMEM((GB,), jnp.int32),
          pltpu.VMEM((GB,), jnp.int32),
          pltpu.VMEM((GB, H), jnp.float32),
          pltpu.VMEM((GB, H), jnp.float32),
          pltpu.VMEM((GB, H), jnp.float32),
          pltpu.VMEM((GB, H), jnp.float32),
          pltpu.SemaphoreType.DMA,
          pltpu.SemaphoreType.DMA,
          pltpu.SemaphoreType.DMA,
          pltpu.SemaphoreType.DMA,
      ],
      compiler_params=pltpu.CompilerParams(needs_layout_passes=False),
  )
  def k(ent_hbm, rel_hbm, pk_hbm, dst_hbm, out_hbm,
        dstv, acc, m_l, s_l, cnt_s,
        pblk0, dblk0, pblk1, dblk1,
        lpk0, ldst0, lpk1, ldst1,
        sidx0, ridx0, sidx1, ridx1,
        srows0, rrows0, srows1, rrows1,
        bsem0, bsem1, gsem0, gsem1):
    wid = lax.axis_index("s") * 2 + lax.axis_index("c")
    zi16 = jnp.zeros((16,), jnp.int32)
    zf16 = jnp.zeros((16,), jnp.float32)

    def issue_blk(b, pb, db, sem):
      bo = b * EB
      pltpu.async_copy(pk_hbm.at[pl.ds(bo, EB)], pb, sem)
      pltpu.async_copy(dst_hbm.at[pl.ds(bo, EB)], db, sem)

    def wait_blk(b, pb, db, sem):
      bo = b * EB
      pltpu.make_async_copy(pk_hbm.at[pl.ds(bo, EB)], pb, sem).wait()
      pltpu.make_async_copy(dst_hbm.at[pl.ds(bo, EB)], db, sem).wait()

    def unpack(lpk, base, sidx, ridx):
      for u in range(GB // 16):
        v = lpk[pl.ds(base + 16 * u, 16)]
        sidx[pl.ds(16 * u, 16)] = v & SRC_MASK
        ridx[pl.ds(16 * u, 16)] = lax.shift_right_logical(v, SRC_BITS)

    def issue_gather(sidx, ridx, srows, rrows, gsem):
      pltpu.async_copy(ent_hbm.at[sidx], srows, gsem)
      pltpu.async_copy(rel_hbm.at[ridx], rrows, gsem)

    def wait_gather(sidx, ridx, srows, rrows, gsem):
      pltpu.make_async_copy(ent_hbm.at[sidx], srows, gsem).wait()
      pltpu.make_async_copy(rel_hbm.at[ridx], rrows, gsem).wait()

    def chunk_body(c, _):
      lo = (wid * NCHUNK + c) * NN
      pltpu.sync_copy(ent_hbm.at[pl.ds(lo, NN), :], dstv)

      def zrow(i, _):
        for t in range(NSL):
          acc[i, pl.ds(16 * t, 16)] = zf16
        m_l[i] = jnp.float32(-3.4e38)
        s_l[i] = jnp.float32(0.0)
        return 0
      lax.fori_loop(0, NN, zrow, 0)

      def scan(pb, db, lpk, ldstl, slot):
        U = 4

        def body(jv, cnt):
          incls = []
          pcs = []
          vdls = []
          msks = []
          for u in range(U):
            sl = pl.ds((jv * U + u) * 16, 16)
            vdl = db[sl] - lo
            msk = vdl.astype(jnp.uint32) < jnp.uint32(NN)
            incl = plsc.cumsum(jnp.where(msk, 1, 0).astype(jnp.int32))
            vdls.append(vdl)
            msks.append(msk)
            incls.append(incl)
            pcs.append(incl[15])
          c = cnt
          for u in range(U):
            sl = pl.ds((jv * U + u) * 16, 16)
            pos = c + incls[u] - 1
            plsc.store_scatter(lpk, [pos], pb[sl], mask=msks[u])
            plsc.store_scatter(ldstl, [pos], vdls[u], mask=msks[u])
            c = c + pcs[u]
          return c
        cnt = lax.fori_loop(0, EB // (16 * U), body, jnp.int32(0))
        lpk[pl.ds(cnt, 16)] = zi16
        lpk[pl.ds(cnt + 16, 16)] = zi16
        cnt_s[slot] = cnt

      def edge_loop(ne, base, ldstl, srows, rrows):
        def edge_body(j, _):
          ldj = ldstl[pl.ds(base + j, 16)][0]
          dp = [jnp.zeros((16,), jnp.float32) for _ in range(4)]
          for t in range(NSL):
            hs = pl.ds(16 * t, 16)
            dp[t % 4] = dp[t % 4] + (srows[j, hs] + rrows[j, hs]) * dstv[ldj, hs]
          norm = jnp.sum((dp[0] + dp[1]) + (dp[2] + dp[3]))
          m_old = m_l[ldj]
          s_old = s_l[ldj]
          m_new = jnp.maximum(m_old, norm)
          w16 = jnp.exp(jnp.full((16,), norm - m_new, jnp.float32))
          corr16 = jnp.exp(jnp.full((16,), m_old - m_new, jnp.float32))
          s_l[ldj] = s_old * corr16[0] + w16[0]
          m_l[ldj] = m_new
          for t in range(NSL):
            hs = pl.ds(16 * t, 16)
            acc[ldj, hs] = (acc[ldj, hs] * corr16
                            + (srows[j, hs] + rrows[j, hs]) * w16)
          return 0
        lax.fori_loop(0, ne, edge_body, 0)

      def process(lpk, ldstl, sidx, ridx, srows, rrows, gsem, slot):
        cnt = cnt_s[slot]
        wait_gather(sidx, ridx, srows, rrows, gsem)
        edge_loop(jnp.minimum(cnt, GB), 0, ldstl, srows, rrows)
        nbat = (cnt + (GB - 1)) // GB

        def extra(bb, _):
          base = bb * GB
          unpack(lpk, base, sidx, ridx)
          issue_gather(sidx, ridx, srows, rrows, gsem)
          wait_gather(sidx, ridx, srows, rrows, gsem)
          edge_loop(jnp.minimum(cnt - base, GB), base, ldstl, srows, rrows)
          return 0
        lax.fori_loop(1, nbat, extra, 0)

      def scan_and_issue(pb, db, lpk, ldstl, sidx, ridx, srows, rrows,
                         gsem, slot):
        scan(pb, db, lpk, ldstl, slot)
        unpack(lpk, 0, sidx, ridx)
        issue_gather(sidx, ridx, srows, rrows, gsem)

      issue_blk(0, pblk0, dblk0, bsem0)
      issue_blk(1, pblk1, dblk1, bsem1)
      wait_blk(0, pblk0, dblk0, bsem0)
      scan_and_issue(pblk0, dblk0, lpk0, ldst0, sidx0, ridx0,
                     srows0, rrows0, gsem0, 0)
      issue_blk(2, pblk0, dblk0, bsem0)

      def pair_body(i, _):
        b0 = 2 * i
        wait_blk(b0 + 1, pblk1, dblk1, bsem1)
        scan_and_issue(pblk1, dblk1, lpk1, ldst1, sidx1, ridx1,
                       srows1, rrows1, gsem1, 1)

        @pl.when(b0 + 3 < n_blocks)
        def _():
          issue_blk(b0 + 3, pblk1, dblk1, bsem1)

        process(lpk0, ldst0, sidx0, ridx0, srows0, rrows0, gsem0, 0)

        @pl.when(b0 + 2 < n_blocks)
        def _():
          wait_blk(b0 + 2, pblk0, dblk0, bsem0)
          scan_and_issue(pblk0, dblk0, lpk0, ldst0, sidx0, ridx0,
                         srows0, rrows0, gsem0, 0)

          @pl.when(b0 + 4 < n_blocks)
          def _():
            issue_blk(b0 + 4, pblk0, dblk0, bsem0)

        process(lpk1, ldst1, sidx1, ridx1, srows1, rrows1, gsem1, 1)
        return 0
      lax.fori_loop(0, npairs, pair_body, 0)

      def fin(i, _):
        sv = jnp.full((16,), s_l[i], jnp.float32) + jnp.float32(1e-16)
        invv = jnp.float32(1.0) / sv
        for t in range(NSL):
          hs = pl.ds(16 * t, 16)
          acc[i, hs] = acc[i, hs] * invv
        return 0
      lax.fori_loop(0, NN, fin, 0)
      pltpu.sync_copy(acc, out_hbm.at[pl.ds(lo, NN), :])
      return 0
    lax.fori_loop(0, NCHUNK, chunk_body, 0)

  return k


def _mm_tanh_body(neigh_ref, w_ref, out_ref):
  out_ref[...] = jnp.tanh(
      jnp.dot(neigh_ref[...], w_ref[...], preferred_element_type=jnp.float32))


def kernel(ent_emb, rel_emb, edge_index, rel_id, neigh_w):
  src = edge_index[0].astype(jnp.int32)
  dst = edge_index[1].astype(jnp.int32)
  rel = rel_id.astype(jnp.int32)
  packed = src | (rel << SRC_BITS)
  n_edges = src.shape[0]

  ent_pad = jnp.concatenate(
      [ent_emb, jnp.zeros((NPAD - N_ENT, H), jnp.float32)], axis=0)

  neigh = _make_sc_neigh(n_edges)(ent_pad, rel_emb, packed, dst)
  neigh = neigh[:N_ENT]

  blk = 1000
  out = pl.pallas_call(
      _mm_tanh_body,
      grid=(N_ENT // blk,),
      in_specs=[
          pl.BlockSpec((blk, H), lambda i: (i, 0)),
          pl.BlockSpec((H, H), lambda i: (0, 0)),
      ],
      out_specs=pl.BlockSpec((blk, H), lambda i: (i, 0)),
      out_shape=jax.ShapeDtypeStruct((N_ENT, H), jnp.float32),
  )(neigh, neigh_w)
  return out
